# baseline (speedup 1.0000x reference)
import functools

import jax
import jax.numpy as jnp
from jax import lax
from jax.experimental import pallas as pl
from jax.experimental.pallas import tpu as pltpu
from jax.experimental.pallas import tpu_sc as plsc

_CHUNK = 128


@functools.lru_cache(maxsize=None)
def _make_gather(V, D, B):
    info = plsc.get_sparse_core_info()
    nc, ns = info.num_cores, info.num_subcores
    nw = nc * ns
    b_per_w = B // nw
    n_chunks = b_per_w // _CHUNK
    mesh = plsc.VectorSubcoreMesh(core_axis_name="c", subcore_axis_name="s")

    @functools.partial(
        pl.kernel,
        mesh=mesh,
        out_type=jax.ShapeDtypeStruct((B, D), jnp.float32),
        scratch_types=[
            pltpu.VMEM((n_chunks, _CHUNK), jnp.int32),
            pltpu.VMEM((b_per_w, D), jnp.float32),
            pltpu.SemaphoreType.DMA,
        ],
        compiler_params=pltpu.CompilerParams(use_tc_tiling_on_sc=False),
    )
    def gather_kernel(idx_hbm, table_hbm, out_hbm, idx_v, rows_v, sem):
        wid = lax.axis_index("s") * nc + lax.axis_index("c")
        base = wid * b_per_w
        pltpu.sync_copy(idx_hbm.at[wid], idx_v)
        copies = []
        for j in range(n_chunks):
            copies.append(
                pltpu.async_copy(
                    table_hbm.at[idx_v.at[j]],
                    rows_v.at[pl.ds(j * _CHUNK, _CHUNK)],
                    sem,
                )
            )
        for c in copies:
            c.wait()
        pltpu.sync_copy(rows_v, out_hbm.at[pl.ds(base, b_per_w)])

    return gather_kernel, nw


def kernel(x, weight):
    (B,) = x.shape
    V, D = weight.shape
    fn, nw = _make_gather(V, D, B)
    idx = x.astype(jnp.int32).reshape(nw, (B // nw) // _CHUNK, _CHUNK)
    return fn(idx, weight)


# speedup vs baseline: 1.9534x; 1.9534x over previous
import functools

import jax
import jax.numpy as jnp
from jax import lax
from jax.experimental import pallas as pl
from jax.experimental.pallas import tpu as pltpu
from jax.experimental.pallas import tpu_sc as plsc


@functools.lru_cache(maxsize=None)
def _make_gather(V, D, B):
    info = plsc.get_sparse_core_info()
    nc, ns = info.num_cores, info.num_subcores
    nw = nc * ns
    b_per_w = B // nw
    n_groups = b_per_w // 16
    nband = D // 8
    mesh = plsc.VectorSubcoreMesh(core_axis_name="c", subcore_axis_name="s")

    @functools.partial(
        pl.kernel,
        mesh=mesh,
        out_type=jax.ShapeDtypeStruct((B * D,), jnp.float32),
        scratch_types=[
            pltpu.VMEM((b_per_w,), jnp.int32),
            pltpu.VMEM((2, nband, 8, 128), jnp.float32),
            pltpu.VMEM((b_per_w * D,), jnp.float32),
            pltpu.SemaphoreType.DMA,
        ],
        compiler_params=pltpu.CompilerParams(
            disable_bounds_checks=True, needs_layout_passes=False
        ),
    )
    def gather_kernel(idx_hbm, wt_hbm, out_hbm, idx_v, stage_v, out_v, sem):
        wid = lax.axis_index("s") * nc + lax.axis_index("c")
        base = wid * b_per_w
        pltpu.sync_copy(idx_hbm.at[pl.ds(base, b_per_w)], idx_v)

        lane = lax.iota(jnp.int32, 16)
        band_idx = [(16 * g + lane) >> 3 for g in range(D // 16)]
        sub_idx = [(16 * g + lane) & 7 for g in range(D // 16)]

        def fetch(i, buf):
            c0 = pl.multiple_of((i >> 7) * 128, 128)
            for r in range(nband):
                pltpu.async_copy(
                    wt_hbm.at[pl.ds(8 * r, 8), pl.ds(c0, 128)],
                    stage_v.at[buf, r],
                    sem,
                )

        def drain(buf):
            for r in range(nband):
                pltpu.make_async_copy(
                    wt_hbm.at[pl.ds(0, 8), pl.ds(0, 128)],
                    stage_v.at[buf, r],
                    sem,
                ).wait()

        def extract(l, buf, off):
            lvec = jnp.full((16,), l, jnp.int32)
            for g2 in range(D // 16):
                vals = plsc.load_gather(
                    stage_v.at[buf], [band_idx[g2], sub_idx[g2], lvec]
                )
                out_v[pl.ds(pl.multiple_of(off + 16 * g2, 16), 16)] = vals

        def body(g, l_prev):
            goff = jnp.minimum(g, n_groups - 1) * 16
            v16 = idx_v[pl.ds(pl.multiple_of(goff, 16), 16)]
            lp = l_prev
            for lane_j in range(16):
                j = g * 16 + lane_j
                cur = lane_j & 1
                prv = 1 - cur

                @pl.when(j < b_per_w)
                def _():
                    fetch(v16[lane_j], cur)

                @pl.when(jnp.logical_and(j >= 1, j <= b_per_w))
                def _():
                    drain(prv)
                    extract(lp, prv, (j - 1) * D)

                lp = v16[lane_j] & 127
            return lp

        lax.fori_loop(0, n_groups + 1, body, jnp.int32(0))
        pltpu.sync_copy(out_v, out_hbm.at[pl.ds(base * D, b_per_w * D)])

    return gather_kernel, nw


def kernel(x, weight):
    (B,) = x.shape
    V, D = weight.shape
    fn, nw = _make_gather(V, D, B)
    idx = x.astype(jnp.int32)
    flat = fn(idx, weight.T)
    return flat.reshape(B, D)
